# Initial kernel scaffold; baseline (speedup 1.0000x reference)
import jax
import jax.numpy as jnp
from jax.experimental import pallas as pl


def kernel(p, x, o, neighbor_idx, W, gamma, beta):
    raise NotImplementedError("write your pallas kernel here")


# speedup vs baseline: 2.5878x; 2.5878x over previous
import functools

import jax
import jax.numpy as jnp
from jax import lax
from jax.experimental import pallas as pl
from jax.experimental.pallas import tpu as pltpu
from jax.experimental.pallas import tpu_sc as plsc

N = 10000
K = 32
C = 128
OUT = 128
EPS = 1e-5

NW = 32
PB = 16
PPW = 320
NPAD = NW * PPW
NB = PPW // PB
KC = 8
NCH = K // KC
LANES = 16
NEG = float("-inf")


def _sc_gather_max(idx_t, x, px, py, pz):
    mesh = plsc.VectorSubcoreMesh(core_axis_name="c", subcore_axis_name="s")

    @functools.partial(
        pl.kernel,
        mesh=mesh,
        compiler_params=pltpu.CompilerParams(needs_layout_passes=False),
        out_type=[
            jax.ShapeDtypeStruct((NPAD, C), jnp.float32),
            jax.ShapeDtypeStruct((NW, 3, PPW), jnp.float32),
        ],
        scratch_types=[
            pltpu.VMEM((NB * K * PB,), jnp.int32),
            pltpu.VMEM((NPAD,), jnp.float32),
            pltpu.VMEM((NPAD,), jnp.float32),
            pltpu.VMEM((NPAD,), jnp.float32),
            pltpu.VMEM((KC * PB, C), jnp.float32),
            pltpu.VMEM((PB, C), jnp.float32),
            pltpu.VMEM((3, PPW), jnp.float32),
            pltpu.SemaphoreType.DMA,
        ],
    )
    def sc_kernel(idxt_hbm, x_hbm, px_hbm, py_hbm, pz_hbm,
                  maxx_hbm, maxp_hbm,
                  idxt_v, px_v, py_v, pz_v, xg, mx, mpt, sem):
        wid = lax.axis_index("s") * 2 + lax.axis_index("c")
        base_pt = wid * PPW
        pltpu.sync_copy(idxt_hbm.at[pl.ds(wid * (NB * K * PB), NB * K * PB)],
                        idxt_v)
        pltpu.sync_copy(px_hbm, px_v)
        pltpu.sync_copy(py_hbm, py_v)
        pltpu.sync_copy(pz_hbm, pz_v)

        def p_block(b, carry):
            boff = b * (K * PB)
            iv = idxt_v[pl.ds(pl.multiple_of(boff, PB), LANES)]
            ax = plsc.load_gather(px_v, [iv])
            ay = plsc.load_gather(py_v, [iv])
            az = plsc.load_gather(pz_v, [iv])
            for k in range(1, K):
                iv = idxt_v[pl.ds(pl.multiple_of(boff + k * PB, PB), LANES)]
                ax = jnp.maximum(ax, plsc.load_gather(px_v, [iv]))
                ay = jnp.maximum(ay, plsc.load_gather(py_v, [iv]))
                az = jnp.maximum(az, plsc.load_gather(pz_v, [iv]))
            off = pl.multiple_of(b * PB, PB)
            mpt[0, pl.ds(off, LANES)] = ax
            mpt[1, pl.ds(off, LANES)] = ay
            mpt[2, pl.ds(off, LANES)] = az
            return carry

        lax.fori_loop(0, NB, p_block, 0, unroll=False)
        pltpu.sync_copy(mpt, maxp_hbm.at[wid])

        neg = jnp.full((LANES,), NEG, jnp.float32)

        def chunk(t, carry):
            b = t // NCH
            kc = t % NCH

            @pl.when(kc == 0)
            def _init():
                for l in range(PB):
                    for c in range(C // LANES):
                        mx[l, pl.ds(c * LANES, LANES)] = neg

            coff = pl.multiple_of(t * (KC * PB), KC * PB)
            pltpu.async_copy(x_hbm.at[idxt_v.at[pl.ds(coff, KC * PB)]],
                             xg, sem).wait()
            for l in range(PB):
                for c in range(C // LANES):
                    acc = mx[l, pl.ds(c * LANES, LANES)]
                    for k in range(KC):
                        acc = jnp.maximum(acc, xg[k * PB + l, pl.ds(c * LANES, LANES)])
                    mx[l, pl.ds(c * LANES, LANES)] = acc

            @pl.when(kc == NCH - 1)
            def _flush():
                pltpu.sync_copy(mx, maxx_hbm.at[pl.ds(base_pt + b * PB, PB)])

            return carry

        lax.fori_loop(0, NB * NCH, chunk, 0, unroll=False)

    return sc_kernel(idx_t, x, px, py, pz)


def _tc_mlp_bn(maxp_t, p_t, maxx, Wx, Wf, gamma, beta):
    def body(maxpt_ref, pt_ref, maxx_ref, wx_ref, wf_ref, g_ref, b_ref, out_ref):
        h = jnp.dot(maxx_ref[...], wf_ref[...], preferred_element_type=jnp.float32)
        for cc in range(3):
            d = (maxpt_ref[cc, :] - pt_ref[cc, :]).reshape(N, 1)
            h = h + d * wx_ref[cc, :].reshape(1, OUT)
        mean = jnp.mean(h, axis=0, keepdims=True)
        hc = h - mean
        var = jnp.mean(hc * hc, axis=0, keepdims=True)
        y = hc * lax.rsqrt(var + EPS) * g_ref[...] + b_ref[...]
        out_ref[...] = jnp.maximum(y, 0.0)

    return pl.pallas_call(
        body,
        out_shape=jax.ShapeDtypeStruct((N, OUT), jnp.float32),
    )(maxp_t, p_t, maxx, Wx, Wf, gamma, beta)


def kernel(p, x, o, neighbor_idx, W, gamma, beta):
    del o
    idx = neighbor_idx.astype(jnp.int32)
    idx_pad = jnp.pad(idx, ((0, NPAD - N), (0, 0)))
    idx_t = idx_pad.reshape(NPAD // PB, PB, K).transpose(0, 2, 1).reshape(-1)
    p_t = p.T
    p_pad = jnp.pad(p_t, ((0, 0), (0, NPAD - N)))
    maxx, maxp = _sc_gather_max(idx_t, x, p_pad[0], p_pad[1], p_pad[2])
    maxp_t = maxp.transpose(1, 0, 2).reshape(3, NPAD)[:, :N]
    return _tc_mlp_bn(maxp_t, p_t, maxx[:N], W[:3], W[3:],
                      gamma[None, :], beta[None, :])


# speedup vs baseline: 3.5145x; 1.3581x over previous
import functools

import jax
import jax.numpy as jnp
from jax import lax
from jax.experimental import pallas as pl
from jax.experimental.pallas import tpu as pltpu
from jax.experimental.pallas import tpu_sc as plsc

N = 10000
K = 32
C = 128
OUT = 128
EPS = 1e-5

NW = 32
PB = 16
PPW = 320
NPAD = NW * PPW
NB = PPW // PB
KC = 8
NCH = K // KC
LANES = 16
NEG = float("-inf")


def _sc_gather_max(idx_t, x, px, py, pz):
    mesh = plsc.VectorSubcoreMesh(core_axis_name="c", subcore_axis_name="s")

    @functools.partial(
        pl.kernel,
        mesh=mesh,
        compiler_params=pltpu.CompilerParams(needs_layout_passes=False),
        out_type=[
            jax.ShapeDtypeStruct((NPAD, C), jnp.float32),
            jax.ShapeDtypeStruct((NW, 3, PPW), jnp.float32),
        ],
        scratch_types=[
            pltpu.VMEM((NB * K * PB,), jnp.int32),
            pltpu.VMEM((NPAD,), jnp.float32),
            pltpu.VMEM((NPAD,), jnp.float32),
            pltpu.VMEM((NPAD,), jnp.float32),
            pltpu.VMEM((KC * PB, C), jnp.float32),
            pltpu.VMEM((KC * PB, C), jnp.float32),
            pltpu.VMEM((PB, C), jnp.float32),
            pltpu.VMEM((3, PPW), jnp.float32),
            pltpu.SemaphoreType.DMA,
            pltpu.SemaphoreType.DMA,
        ],
    )
    def sc_kernel(idxt_hbm, x_hbm, px_hbm, py_hbm, pz_hbm,
                  maxx_hbm, maxp_hbm,
                  idxt_v, px_v, py_v, pz_v, xg0, xg1, mx, mpt, sem0, sem1):
        wid = lax.axis_index("s") * 2 + lax.axis_index("c")
        base_pt = wid * PPW
        pltpu.sync_copy(idxt_hbm.at[pl.ds(wid * (NB * K * PB), NB * K * PB)],
                        idxt_v)
        pltpu.sync_copy(px_hbm, px_v)
        pltpu.sync_copy(py_hbm, py_v)
        pltpu.sync_copy(pz_hbm, pz_v)

        def p_block(b, carry):
            boff = b * (K * PB)
            iv = idxt_v[pl.ds(pl.multiple_of(boff, PB), LANES)]
            ax = plsc.load_gather(px_v, [iv])
            ay = plsc.load_gather(py_v, [iv])
            az = plsc.load_gather(pz_v, [iv])
            for k in range(1, K):
                iv = idxt_v[pl.ds(pl.multiple_of(boff + k * PB, PB), LANES)]
                ax = jnp.maximum(ax, plsc.load_gather(px_v, [iv]))
                ay = jnp.maximum(ay, plsc.load_gather(py_v, [iv]))
                az = jnp.maximum(az, plsc.load_gather(pz_v, [iv]))
            off = pl.multiple_of(b * PB, PB)
            mpt[0, pl.ds(off, LANES)] = ax
            mpt[1, pl.ds(off, LANES)] = ay
            mpt[2, pl.ds(off, LANES)] = az
            return carry

        lax.fori_loop(0, NB, p_block, 0, unroll=False)
        pltpu.sync_copy(mpt, maxp_hbm.at[wid])

        neg = jnp.full((LANES,), NEG, jnp.float32)
        T = NB * NCH
        bufs = ((xg0, sem0), (xg1, sem1))

        def issue(t, xg, sem):
            coff = pl.multiple_of(t * (KC * PB), KC * PB)
            pltpu.async_copy(x_hbm.at[idxt_v.at[pl.ds(coff, KC * PB)]],
                             xg, sem)

        issue(0, xg0, sem0)
        issue(1, xg1, sem1)

        def step(t, xg, sem):
            b = t // NCH
            kc = t % NCH
            pltpu.make_async_copy(x_hbm.at[pl.ds(0, KC * PB)], xg, sem).wait()

            @pl.when(kc == 0)
            def _init():
                for l in range(PB):
                    for c in range(C // LANES):
                        mx[l, pl.ds(c * LANES, LANES)] = neg

            for l in range(PB):
                for c in range(C // LANES):
                    acc = mx[l, pl.ds(c * LANES, LANES)]
                    for k in range(KC):
                        acc = jnp.maximum(acc, xg[k * PB + l, pl.ds(c * LANES, LANES)])
                    mx[l, pl.ds(c * LANES, LANES)] = acc

            @pl.when(kc == NCH - 1)
            def _flush():
                pltpu.sync_copy(mx, maxx_hbm.at[pl.ds(base_pt + b * PB, PB)])

            @pl.when(t + 2 < T)
            def _next():
                issue(t + 2, xg, sem)

        def pair(t2, carry):
            for par in range(2):
                step(t2 * 2 + par, *bufs[par])
            return carry

        lax.fori_loop(0, T // 2, pair, 0, unroll=False)

    return sc_kernel(idx_t, x, px, py, pz)


def _tc_mlp_bn(maxp_t, p_t, maxx, Wx, Wf, gamma, beta):
    def body(maxpt_ref, pt_ref, maxx_ref, wx_ref, wf_ref, g_ref, b_ref, out_ref):
        h = jnp.dot(maxx_ref[...], wf_ref[...], preferred_element_type=jnp.float32)
        for cc in range(3):
            d = (maxpt_ref[cc, :] - pt_ref[cc, :]).reshape(N, 1)
            h = h + d * wx_ref[cc, :].reshape(1, OUT)
        mean = jnp.mean(h, axis=0, keepdims=True)
        hc = h - mean
        var = jnp.mean(hc * hc, axis=0, keepdims=True)
        y = hc * lax.rsqrt(var + EPS) * g_ref[...] + b_ref[...]
        out_ref[...] = jnp.maximum(y, 0.0)

    return pl.pallas_call(
        body,
        out_shape=jax.ShapeDtypeStruct((N, OUT), jnp.float32),
    )(maxp_t, p_t, maxx, Wx, Wf, gamma, beta)


def kernel(p, x, o, neighbor_idx, W, gamma, beta):
    del o
    idx = neighbor_idx.astype(jnp.int32)
    idx_pad = jnp.pad(idx, ((0, NPAD - N), (0, 0)))
    idx_t = idx_pad.reshape(NPAD // PB, PB, K).transpose(0, 2, 1).reshape(-1)
    p_t = p.T
    p_pad = jnp.pad(p_t, ((0, 0), (0, NPAD - N)))
    maxx, maxp = _sc_gather_max(idx_t, x, p_pad[0], p_pad[1], p_pad[2])
    maxp_t = maxp.transpose(1, 0, 2).reshape(3, NPAD)[:, :N]
    return _tc_mlp_bn(maxp_t, p_t, maxx[:N], W[:3], W[3:],
                      gamma[None, :], beta[None, :])


# speedup vs baseline: 3.6239x; 1.0311x over previous
import functools

import jax
import jax.numpy as jnp
from jax import lax
from jax.experimental import pallas as pl
from jax.experimental.pallas import tpu as pltpu
from jax.experimental.pallas import tpu_sc as plsc

N = 10000
K = 32
C = 128
OUT = 128
EPS = 1e-5

NW = 32
PB = 16
PPW = 320
NPAD = NW * PPW
NB = PPW // PB
KC = 8
GPTS = 4
LANES = 16


def _sc_gather_max(idx_flat, idx_t, x, px, py, pz):
    mesh = plsc.VectorSubcoreMesh(core_axis_name="c", subcore_axis_name="s")

    @functools.partial(
        pl.kernel,
        mesh=mesh,
        compiler_params=pltpu.CompilerParams(needs_layout_passes=False),
        out_type=[
            jax.ShapeDtypeStruct((NPAD, C), jnp.float32),
            jax.ShapeDtypeStruct((NW, 3, PPW), jnp.float32),
        ],
        scratch_types=[
            pltpu.VMEM((PPW * K,), jnp.int32),
            pltpu.VMEM((NB * K * PB,), jnp.int32),
            pltpu.VMEM((NPAD,), jnp.float32),
            pltpu.VMEM((NPAD,), jnp.float32),
            pltpu.VMEM((NPAD,), jnp.float32),
            pltpu.VMEM((KC * PB, C), jnp.float32),
            pltpu.VMEM((KC * PB, C), jnp.float32),
            pltpu.VMEM((GPTS, C), jnp.float32),
            pltpu.VMEM((GPTS, C), jnp.float32),
            pltpu.VMEM((3, PPW), jnp.float32),
            pltpu.SemaphoreType.DMA,
            pltpu.SemaphoreType.DMA,
            pltpu.SemaphoreType.DMA,
            pltpu.SemaphoreType.DMA,
        ],
    )
    def sc_kernel(idx_hbm, idxt_hbm, x_hbm, px_hbm, py_hbm, pz_hbm,
                  maxx_hbm, maxp_hbm,
                  idx_v, idxt_v, px_v, py_v, pz_v, xg0, xg1, mx0, mx1, mpt,
                  gsem0, gsem1, fsem0, fsem1):
        wid = lax.axis_index("s") * 2 + lax.axis_index("c")
        base_pt = wid * PPW
        pltpu.sync_copy(idx_hbm.at[pl.ds(wid * (PPW * K), PPW * K)], idx_v)
        pltpu.sync_copy(idxt_hbm.at[pl.ds(wid * (NB * K * PB), NB * K * PB)],
                        idxt_v)
        pltpu.sync_copy(px_hbm, px_v)
        pltpu.sync_copy(py_hbm, py_v)
        pltpu.sync_copy(pz_hbm, pz_v)

        def p_block(b, carry):
            boff = b * (K * PB)
            iv = idxt_v[pl.ds(pl.multiple_of(boff, PB), LANES)]
            ax = plsc.load_gather(px_v, [iv])
            ay = plsc.load_gather(py_v, [iv])
            az = plsc.load_gather(pz_v, [iv])
            for k in range(1, K):
                iv = idxt_v[pl.ds(pl.multiple_of(boff + k * PB, PB), LANES)]
                ax = jnp.maximum(ax, plsc.load_gather(px_v, [iv]))
                ay = jnp.maximum(ay, plsc.load_gather(py_v, [iv]))
                az = jnp.maximum(az, plsc.load_gather(pz_v, [iv]))
            off = pl.multiple_of(b * PB, PB)
            mpt[0, pl.ds(off, LANES)] = ax
            mpt[1, pl.ds(off, LANES)] = ay
            mpt[2, pl.ds(off, LANES)] = az
            return carry

        lax.fori_loop(0, NB, p_block, 0, unroll=False)
        pltpu.sync_copy(mpt, maxp_hbm.at[wid])

        T = PPW // GPTS
        bufs = ((xg0, gsem0, mx0, fsem0), (xg1, gsem1, mx1, fsem1))

        def issue(t, xg, gsem):
            coff = pl.multiple_of(t * (GPTS * K), GPTS * K)
            pltpu.async_copy(x_hbm.at[idx_v.at[pl.ds(coff, GPTS * K)]],
                             xg, gsem)

        issue(0, xg0, gsem0)
        issue(1, xg1, gsem1)

        def step(t, xg, gsem, mxb, fsem):
            pltpu.make_async_copy(x_hbm.at[pl.ds(0, GPTS * K)], xg, gsem).wait()

            @pl.when(t >= 2)
            def _drain_flush():
                pltpu.make_async_copy(maxx_hbm.at[pl.ds(0, GPTS)], mxb,
                                      fsem).wait()

            for pt in range(GPTS):
                for c in range(C // LANES):
                    vals = [xg[pt * K + k, pl.ds(c * LANES, LANES)]
                            for k in range(K)]
                    while len(vals) > 1:
                        vals = [jnp.maximum(vals[i], vals[i + 1])
                                for i in range(0, len(vals), 2)]
                    mxb[pt, pl.ds(c * LANES, LANES)] = vals[0]
            pltpu.async_copy(mxb, maxx_hbm.at[pl.ds(base_pt + t * GPTS, GPTS)],
                             fsem)

            @pl.when(t + 2 < T)
            def _next():
                issue(t + 2, xg, gsem)

        def pair(t2, carry):
            for par in range(2):
                step(t2 * 2 + par, *bufs[par])
            return carry

        lax.fori_loop(0, T // 2, pair, 0, unroll=False)
        pltpu.make_async_copy(maxx_hbm.at[pl.ds(0, GPTS)], mx0, fsem0).wait()
        pltpu.make_async_copy(maxx_hbm.at[pl.ds(0, GPTS)], mx1, fsem1).wait()

    return sc_kernel(idx_flat, idx_t, x, px, py, pz)


def _tc_mlp_bn(maxp_t, p_t, maxx, Wx, Wf, gamma, beta):
    def body(maxpt_ref, pt_ref, maxx_ref, wx_ref, wf_ref, g_ref, b_ref, out_ref):
        h = jnp.dot(maxx_ref[...], wf_ref[...], preferred_element_type=jnp.float32)
        for cc in range(3):
            d = (maxpt_ref[cc, :] - pt_ref[cc, :]).reshape(N, 1)
            h = h + d * wx_ref[cc, :].reshape(1, OUT)
        mean = jnp.mean(h, axis=0, keepdims=True)
        hc = h - mean
        var = jnp.mean(hc * hc, axis=0, keepdims=True)
        y = hc * lax.rsqrt(var + EPS) * g_ref[...] + b_ref[...]
        out_ref[...] = jnp.maximum(y, 0.0)

    return pl.pallas_call(
        body,
        out_shape=jax.ShapeDtypeStruct((N, OUT), jnp.float32),
    )(maxp_t, p_t, maxx, Wx, Wf, gamma, beta)


def kernel(p, x, o, neighbor_idx, W, gamma, beta):
    del o
    idx = neighbor_idx.astype(jnp.int32)
    idx_pad = jnp.pad(idx, ((0, NPAD - N), (0, 0)))
    idx_flat = idx_pad.reshape(-1)
    idx_t = idx_pad.reshape(NPAD // PB, PB, K).transpose(0, 2, 1).reshape(-1)
    p_t = p.T
    p_pad = jnp.pad(p_t, ((0, 0), (0, NPAD - N)))
    maxx, maxp = _sc_gather_max(idx_flat, idx_t, x,
                                p_pad[0], p_pad[1], p_pad[2])
    maxp_t = maxp.transpose(1, 0, 2).reshape(3, NPAD)[:, :N]
    return _tc_mlp_bn(maxp_t, p_t, maxx[:N], W[:3], W[3:],
                      gamma[None, :], beta[None, :])


# speedup vs baseline: 4.0304x; 1.1121x over previous
import functools

import jax
import jax.numpy as jnp
from jax import lax
from jax.experimental import pallas as pl
from jax.experimental.pallas import tpu as pltpu
from jax.experimental.pallas import tpu_sc as plsc

N = 10000
K = 32
C = 128
OUT = 128
EPS = 1e-5

NW = 32
PB = 16
PPW = 320
NPAD = NW * PPW
NB = PPW // PB
KC = 8
GPTS = 4
LANES = 16


def _sc_gather_max(idx_flat, idx_t, x, px, py, pz):
    mesh = plsc.VectorSubcoreMesh(core_axis_name="c", subcore_axis_name="s")

    @functools.partial(
        pl.kernel,
        mesh=mesh,
        compiler_params=pltpu.CompilerParams(needs_layout_passes=False),
        out_type=[
            jax.ShapeDtypeStruct((NPAD, C), jnp.float32),
            jax.ShapeDtypeStruct((NW, 3, PPW), jnp.float32),
        ],
        scratch_types=[
            pltpu.VMEM((PPW * K,), jnp.int32),
            pltpu.VMEM((NB * K * PB,), jnp.int32),
            pltpu.VMEM((NPAD,), jnp.float32),
            pltpu.VMEM((NPAD,), jnp.float32),
            pltpu.VMEM((NPAD,), jnp.float32),
            pltpu.VMEM((GPTS * K, C), jnp.float32),
            pltpu.VMEM((GPTS * K, C), jnp.float32),
            pltpu.VMEM((GPTS * K, C), jnp.float32),
            pltpu.VMEM((GPTS * K, C), jnp.float32),
            pltpu.VMEM((GPTS, C), jnp.float32),
            pltpu.VMEM((GPTS, C), jnp.float32),
            pltpu.VMEM((GPTS, C), jnp.float32),
            pltpu.VMEM((GPTS, C), jnp.float32),
            pltpu.VMEM((3, PPW), jnp.float32),
            pltpu.SemaphoreType.DMA,
            pltpu.SemaphoreType.DMA,
            pltpu.SemaphoreType.DMA,
            pltpu.SemaphoreType.DMA,
            pltpu.SemaphoreType.DMA,
            pltpu.SemaphoreType.DMA,
            pltpu.SemaphoreType.DMA,
            pltpu.SemaphoreType.DMA,
        ],
    )
    def sc_kernel(idx_hbm, idxt_hbm, x_hbm, px_hbm, py_hbm, pz_hbm,
                  maxx_hbm, maxp_hbm,
                  idx_v, idxt_v, px_v, py_v, pz_v, xg0, xg1, xg2, xg3,
                  mx0, mx1, mx2, mx3, mpt,
                  gsem0, gsem1, gsem2, gsem3, fsem0, fsem1, fsem2, fsem3):
        wid = lax.axis_index("s") * 2 + lax.axis_index("c")
        base_pt = wid * PPW
        pltpu.sync_copy(idx_hbm.at[pl.ds(wid * (PPW * K), PPW * K)], idx_v)
        pltpu.sync_copy(idxt_hbm.at[pl.ds(wid * (NB * K * PB), NB * K * PB)],
                        idxt_v)
        pltpu.sync_copy(px_hbm, px_v)
        pltpu.sync_copy(py_hbm, py_v)
        pltpu.sync_copy(pz_hbm, pz_v)

        def p_block(b, carry):
            boff = b * (K * PB)
            iv = idxt_v[pl.ds(pl.multiple_of(boff, PB), LANES)]
            ax = plsc.load_gather(px_v, [iv])
            ay = plsc.load_gather(py_v, [iv])
            az = plsc.load_gather(pz_v, [iv])
            for k in range(1, K):
                iv = idxt_v[pl.ds(pl.multiple_of(boff + k * PB, PB), LANES)]
                ax = jnp.maximum(ax, plsc.load_gather(px_v, [iv]))
                ay = jnp.maximum(ay, plsc.load_gather(py_v, [iv]))
                az = jnp.maximum(az, plsc.load_gather(pz_v, [iv]))
            off = pl.multiple_of(b * PB, PB)
            mpt[0, pl.ds(off, LANES)] = ax
            mpt[1, pl.ds(off, LANES)] = ay
            mpt[2, pl.ds(off, LANES)] = az
            return carry

        lax.fori_loop(0, NB, p_block, 0, unroll=False)
        pltpu.sync_copy(mpt, maxp_hbm.at[wid])

        T = PPW // GPTS
        NBUF = 4
        bufs = ((xg0, gsem0, mx0, fsem0), (xg1, gsem1, mx1, fsem1),
                (xg2, gsem2, mx2, fsem2), (xg3, gsem3, mx3, fsem3))

        def issue(t, xg, gsem):
            coff = pl.multiple_of(t * (GPTS * K), GPTS * K)
            pltpu.async_copy(x_hbm.at[idx_v.at[pl.ds(coff, GPTS * K)]],
                             xg, gsem)

        for i in range(NBUF):
            issue(i, bufs[i][0], bufs[i][1])

        def step(t, xg, gsem, mxb, fsem):
            pltpu.make_async_copy(x_hbm.at[pl.ds(0, GPTS * K)], xg, gsem).wait()

            @pl.when(t >= NBUF)
            def _drain_flush():
                pltpu.make_async_copy(maxx_hbm.at[pl.ds(0, GPTS)], mxb,
                                      fsem).wait()

            def col(c, carry):
                off = pl.multiple_of(c * LANES, LANES)
                for pt in range(GPTS):
                    vals = [xg[pt * K + k, pl.ds(off, LANES)]
                            for k in range(K)]
                    while len(vals) > 1:
                        vals = [jnp.maximum(vals[i], vals[i + 1])
                                for i in range(0, len(vals), 2)]
                    mxb[pt, pl.ds(off, LANES)] = vals[0]
                return carry

            lax.fori_loop(0, C // LANES, col, 0, unroll=False)
            pltpu.async_copy(mxb, maxx_hbm.at[pl.ds(base_pt + t * GPTS, GPTS)],
                             fsem)

            @pl.when(t + NBUF < T)
            def _next():
                issue(t + NBUF, xg, gsem)

        def rotation(t4, carry):
            for par in range(NBUF):
                step(t4 * NBUF + par, *bufs[par])
            return carry

        lax.fori_loop(0, T // NBUF, rotation, 0, unroll=False)
        for i in range(NBUF):
            pltpu.make_async_copy(maxx_hbm.at[pl.ds(0, GPTS)], bufs[i][2],
                                  bufs[i][3]).wait()

    return sc_kernel(idx_flat, idx_t, x, px, py, pz)


def _tc_mlp_bn(maxp_t, p_t, maxx, Wx, Wf, gamma, beta):
    def body(maxpt_ref, pt_ref, maxx_ref, wx_ref, wf_ref, g_ref, b_ref, out_ref):
        h = jnp.dot(maxx_ref[...], wf_ref[...], preferred_element_type=jnp.float32)
        for cc in range(3):
            d = (maxpt_ref[cc, :] - pt_ref[cc, :]).reshape(N, 1)
            h = h + d * wx_ref[cc, :].reshape(1, OUT)
        mean = jnp.mean(h, axis=0, keepdims=True)
        hc = h - mean
        var = jnp.mean(hc * hc, axis=0, keepdims=True)
        y = hc * lax.rsqrt(var + EPS) * g_ref[...] + b_ref[...]
        out_ref[...] = jnp.maximum(y, 0.0)

    return pl.pallas_call(
        body,
        out_shape=jax.ShapeDtypeStruct((N, OUT), jnp.float32),
    )(maxp_t, p_t, maxx, Wx, Wf, gamma, beta)


def kernel(p, x, o, neighbor_idx, W, gamma, beta):
    del o
    idx = neighbor_idx.astype(jnp.int32)
    idx_pad = jnp.pad(idx, ((0, NPAD - N), (0, 0)))
    idx_flat = idx_pad.reshape(-1)
    idx_t = idx_pad.reshape(NPAD // PB, PB, K).transpose(0, 2, 1).reshape(-1)
    p_t = p.T
    p_pad = jnp.pad(p_t, ((0, 0), (0, NPAD - N)))
    maxx, maxp = _sc_gather_max(idx_flat, idx_t, x,
                                p_pad[0], p_pad[1], p_pad[2])
    maxp_t = maxp.transpose(1, 0, 2).reshape(3, NPAD)[:, :N]
    return _tc_mlp_bn(maxp_t, p_t, maxx[:N], W[:3], W[3:],
                      gamma[None, :], beta[None, :])


# speedup vs baseline: 5.9334x; 1.4722x over previous
import functools

import jax
import jax.numpy as jnp
from jax import lax
from jax.experimental import pallas as pl
from jax.experimental.pallas import tpu as pltpu
from jax.experimental.pallas import tpu_sc as plsc

N = 10000
K = 32
C = 128
OUT = 128
EPS = 1e-5

NW = 32
PB = 16
PPW = 320
NPAD = NW * PPW
NB = PPW // PB
KC = 8
GPTS = 4
LANES = 16


def _sc_gather_max(idx_flat, idx_t, x, px, py, pz):
    mesh = plsc.VectorSubcoreMesh(core_axis_name="c", subcore_axis_name="s")

    @functools.partial(
        pl.kernel,
        mesh=mesh,
        compiler_params=pltpu.CompilerParams(
            needs_layout_passes=False, use_tc_tiling_on_sc=False),
        out_type=[
            jax.ShapeDtypeStruct((NPAD, C), jnp.bfloat16),
            jax.ShapeDtypeStruct((NW, 3, PPW), jnp.float32),
        ],
        scratch_types=[
            pltpu.VMEM((PPW * K,), jnp.int32),
            pltpu.VMEM((NB * K * PB,), jnp.int32),
            pltpu.VMEM((NPAD,), jnp.float32),
            pltpu.VMEM((NPAD,), jnp.float32),
            pltpu.VMEM((NPAD,), jnp.float32),
            pltpu.VMEM((GPTS * K, C), jnp.bfloat16),
            pltpu.VMEM((GPTS * K, C), jnp.bfloat16),
            pltpu.VMEM((GPTS * K, C), jnp.bfloat16),
            pltpu.VMEM((GPTS * K, C), jnp.bfloat16),
            pltpu.VMEM((GPTS, C), jnp.bfloat16),
            pltpu.VMEM((GPTS, C), jnp.bfloat16),
            pltpu.VMEM((GPTS, C), jnp.bfloat16),
            pltpu.VMEM((GPTS, C), jnp.bfloat16),
            pltpu.VMEM((3, PPW), jnp.float32),
            pltpu.SemaphoreType.DMA,
            pltpu.SemaphoreType.DMA,
            pltpu.SemaphoreType.DMA,
            pltpu.SemaphoreType.DMA,
            pltpu.SemaphoreType.DMA,
            pltpu.SemaphoreType.DMA,
            pltpu.SemaphoreType.DMA,
            pltpu.SemaphoreType.DMA,
        ],
    )
    def sc_kernel(idx_hbm, idxt_hbm, x_hbm, px_hbm, py_hbm, pz_hbm,
                  maxx_hbm, maxp_hbm,
                  idx_v, idxt_v, px_v, py_v, pz_v, xg0, xg1, xg2, xg3,
                  mx0, mx1, mx2, mx3, mpt,
                  gsem0, gsem1, gsem2, gsem3, fsem0, fsem1, fsem2, fsem3):
        wid = lax.axis_index("s") * 2 + lax.axis_index("c")
        base_pt = wid * PPW
        pltpu.sync_copy(idx_hbm.at[pl.ds(wid * (PPW * K), PPW * K)], idx_v)
        pltpu.sync_copy(idxt_hbm.at[pl.ds(wid * (NB * K * PB), NB * K * PB)],
                        idxt_v)
        pltpu.sync_copy(px_hbm, px_v)
        pltpu.sync_copy(py_hbm, py_v)
        pltpu.sync_copy(pz_hbm, pz_v)

        def p_block(b, carry):
            boff = b * (K * PB)
            iv = idxt_v[pl.ds(pl.multiple_of(boff, PB), LANES)]
            ax = plsc.load_gather(px_v, [iv])
            ay = plsc.load_gather(py_v, [iv])
            az = plsc.load_gather(pz_v, [iv])
            for k in range(1, K):
                iv = idxt_v[pl.ds(pl.multiple_of(boff + k * PB, PB), LANES)]
                ax = jnp.maximum(ax, plsc.load_gather(px_v, [iv]))
                ay = jnp.maximum(ay, plsc.load_gather(py_v, [iv]))
                az = jnp.maximum(az, plsc.load_gather(pz_v, [iv]))
            off = pl.multiple_of(b * PB, PB)
            mpt[0, pl.ds(off, LANES)] = ax
            mpt[1, pl.ds(off, LANES)] = ay
            mpt[2, pl.ds(off, LANES)] = az
            return carry

        lax.fori_loop(0, NB, p_block, 0, unroll=False)
        pltpu.sync_copy(mpt, maxp_hbm.at[wid])

        T = PPW // GPTS
        NBUF = 4
        bufs = ((xg0, gsem0, mx0, fsem0), (xg1, gsem1, mx1, fsem1),
                (xg2, gsem2, mx2, fsem2), (xg3, gsem3, mx3, fsem3))

        def issue(t, xg, gsem):
            coff = pl.multiple_of(t * (GPTS * K), GPTS * K)
            pltpu.async_copy(x_hbm.at[idx_v.at[pl.ds(coff, GPTS * K)]],
                             xg, gsem)

        for i in range(NBUF):
            issue(i, bufs[i][0], bufs[i][1])

        def step(t, xg, gsem, mxb, fsem):
            pltpu.make_async_copy(x_hbm.at[pl.ds(0, GPTS * K)], xg, gsem).wait()

            @pl.when(t >= NBUF)
            def _drain_flush():
                pltpu.make_async_copy(maxx_hbm.at[pl.ds(0, GPTS)], mxb,
                                      fsem).wait()

            def col(c, carry):
                off = pl.multiple_of(c * (2 * LANES), 2 * LANES)
                for pt in range(GPTS):
                    vals = [xg[pt * K + k, pl.ds(off, 2 * LANES)]
                            for k in range(K)]
                    while len(vals) > 1:
                        vals = [jnp.maximum(vals[i], vals[i + 1])
                                for i in range(0, len(vals), 2)]
                    mxb[pt, pl.ds(off, 2 * LANES)] = vals[0]
                return carry

            lax.fori_loop(0, C // (2 * LANES), col, 0, unroll=False)
            pltpu.async_copy(mxb, maxx_hbm.at[pl.ds(base_pt + t * GPTS, GPTS)],
                             fsem)

            @pl.when(t + NBUF < T)
            def _next():
                issue(t + NBUF, xg, gsem)

        def rotation(t4, carry):
            for par in range(NBUF):
                step(t4 * NBUF + par, *bufs[par])
            return carry

        lax.fori_loop(0, T // NBUF, rotation, 0, unroll=False)
        for i in range(NBUF):
            pltpu.make_async_copy(maxx_hbm.at[pl.ds(0, GPTS)], bufs[i][2],
                                  bufs[i][3]).wait()

    return sc_kernel(idx_flat, idx_t, x, px, py, pz)


def _tc_mlp_bn(maxp_t, p_t, maxx, Wx, Wf, gamma, beta):
    def body(maxpt_ref, pt_ref, maxx_ref, wx_ref, wf_ref, g_ref, b_ref, out_ref):
        h = jnp.dot(maxx_ref[...].astype(jnp.float32), wf_ref[...],
                    preferred_element_type=jnp.float32)
        for cc in range(3):
            d = (maxpt_ref[cc, :] - pt_ref[cc, :]).reshape(N, 1)
            h = h + d * wx_ref[cc, :].reshape(1, OUT)
        mean = jnp.mean(h, axis=0, keepdims=True)
        hc = h - mean
        var = jnp.mean(hc * hc, axis=0, keepdims=True)
        y = hc * lax.rsqrt(var + EPS) * g_ref[...] + b_ref[...]
        out_ref[...] = jnp.maximum(y, 0.0)

    return pl.pallas_call(
        body,
        out_shape=jax.ShapeDtypeStruct((N, OUT), jnp.float32),
    )(maxp_t, p_t, maxx, Wx, Wf, gamma, beta)


def kernel(p, x, o, neighbor_idx, W, gamma, beta):
    del o
    idx = neighbor_idx.astype(jnp.int32)
    idx_pad = jnp.pad(idx, ((0, NPAD - N), (0, 0)))
    idx_flat = idx_pad.reshape(-1)
    idx_t = idx_pad.reshape(NPAD // PB, PB, K).transpose(0, 2, 1).reshape(-1)
    p_t = p.T
    p_pad = jnp.pad(p_t, ((0, 0), (0, NPAD - N)))
    maxx, maxp = _sc_gather_max(idx_flat, idx_t, x.astype(jnp.bfloat16),
                                p_pad[0], p_pad[1], p_pad[2])
    maxp_t = maxp.transpose(1, 0, 2).reshape(3, NPAD)[:, :N]
    return _tc_mlp_bn(maxp_t, p_t, maxx[:N], W[:3], W[3:],
                      gamma[None, :], beta[None, :])


# speedup vs baseline: 5.9842x; 1.0086x over previous
import functools

import jax
import jax.numpy as jnp
from jax import lax
from jax.experimental import pallas as pl
from jax.experimental.pallas import tpu as pltpu
from jax.experimental.pallas import tpu_sc as plsc

N = 10000
K = 32
C = 128
OUT = 128
EPS = 1e-5

NW = 32
PB = 16
PPW = 320
NPAD = NW * PPW
NB = PPW // PB
KC = 8
GPTS = 4
LANES = 16


def _sc_gather_max(idx_flat, idx_t, x, px, py, pz):
    mesh = plsc.VectorSubcoreMesh(core_axis_name="c", subcore_axis_name="s")

    @functools.partial(
        pl.kernel,
        mesh=mesh,
        compiler_params=pltpu.CompilerParams(
            needs_layout_passes=False, use_tc_tiling_on_sc=False),
        out_type=[
            jax.ShapeDtypeStruct((NPAD, C), jnp.bfloat16),
            jax.ShapeDtypeStruct((NW, 3, PPW), jnp.float32),
        ],
        scratch_types=[
            pltpu.VMEM((PPW * K,), jnp.int32),
            pltpu.VMEM((NB * K * PB,), jnp.int32),
            pltpu.VMEM((NPAD,), jnp.float32),
            pltpu.VMEM((NPAD,), jnp.float32),
            pltpu.VMEM((NPAD,), jnp.float32),
            pltpu.VMEM((GPTS * K, C), jnp.bfloat16),
            pltpu.VMEM((GPTS * K, C), jnp.bfloat16),
            pltpu.VMEM((GPTS * K, C), jnp.bfloat16),
            pltpu.VMEM((GPTS * K, C), jnp.bfloat16),
            pltpu.VMEM((GPTS, C), jnp.bfloat16),
            pltpu.VMEM((GPTS, C), jnp.bfloat16),
            pltpu.VMEM((GPTS, C), jnp.bfloat16),
            pltpu.VMEM((GPTS, C), jnp.bfloat16),
            pltpu.VMEM((3, PPW), jnp.float32),
            pltpu.SemaphoreType.DMA,
            pltpu.SemaphoreType.DMA,
            pltpu.SemaphoreType.DMA,
            pltpu.SemaphoreType.DMA,
            pltpu.SemaphoreType.DMA,
            pltpu.SemaphoreType.DMA,
            pltpu.SemaphoreType.DMA,
            pltpu.SemaphoreType.DMA,
            pltpu.SemaphoreType.DMA,
        ],
    )
    def sc_kernel(idx_hbm, idxt_hbm, x_hbm, px_hbm, py_hbm, pz_hbm,
                  maxx_hbm, maxp_hbm,
                  idx_v, idxt_v, px_v, py_v, pz_v, xg0, xg1, xg2, xg3,
                  mx0, mx1, mx2, mx3, mpt,
                  gsem0, gsem1, gsem2, gsem3, fsem0, fsem1, fsem2, fsem3,
                  psem):
        wid = lax.axis_index("s") * 2 + lax.axis_index("c")
        base_pt = wid * PPW
        pltpu.sync_copy(idx_hbm.at[pl.ds(wid * (PPW * K), PPW * K)], idx_v)
        pltpu.async_copy(idxt_hbm.at[pl.ds(wid * (NB * K * PB), NB * K * PB)],
                         idxt_v, psem)
        pltpu.async_copy(px_hbm, px_v, psem)
        pltpu.async_copy(py_hbm, py_v, psem)
        pltpu.async_copy(pz_hbm, pz_v, psem)

        T = PPW // GPTS
        NBUF = 4
        bufs = ((xg0, gsem0, mx0, fsem0), (xg1, gsem1, mx1, fsem1),
                (xg2, gsem2, mx2, fsem2), (xg3, gsem3, mx3, fsem3))

        def issue(t, xg, gsem):
            coff = pl.multiple_of(t * (GPTS * K), GPTS * K)
            pltpu.async_copy(x_hbm.at[idx_v.at[pl.ds(coff, GPTS * K)]],
                             xg, gsem)

        for i in range(NBUF):
            issue(i, bufs[i][0], bufs[i][1])

        def step(t, xg, gsem, mxb, fsem):
            pltpu.make_async_copy(x_hbm.at[pl.ds(0, GPTS * K)], xg, gsem).wait()

            @pl.when(t >= NBUF)
            def _drain_flush():
                pltpu.make_async_copy(maxx_hbm.at[pl.ds(0, GPTS)], mxb,
                                      fsem).wait()

            def col(c, carry):
                off = pl.multiple_of(c * (2 * LANES), 2 * LANES)
                for pt in range(GPTS):
                    vals = [xg[pt * K + k, pl.ds(off, 2 * LANES)]
                            for k in range(K)]
                    while len(vals) > 1:
                        vals = [jnp.maximum(vals[i], vals[i + 1])
                                for i in range(0, len(vals), 2)]
                    mxb[pt, pl.ds(off, 2 * LANES)] = vals[0]
                return carry

            lax.fori_loop(0, C // (2 * LANES), col, 0, unroll=False)
            pltpu.async_copy(mxb, maxx_hbm.at[pl.ds(base_pt + t * GPTS, GPTS)],
                             fsem)

            @pl.when(t + NBUF < T)
            def _next():
                issue(t + NBUF, xg, gsem)

        def rotation(t4, carry):
            for par in range(NBUF):
                step(t4 * NBUF + par, *bufs[par])
            return carry

        lax.fori_loop(0, T // NBUF, rotation, 0, unroll=False)

        pltpu.make_async_copy(
            idxt_hbm.at[pl.ds(0, NB * K * PB)], idxt_v, psem).wait()
        pltpu.make_async_copy(px_hbm, px_v, psem).wait()
        pltpu.make_async_copy(py_hbm, py_v, psem).wait()
        pltpu.make_async_copy(pz_hbm, pz_v, psem).wait()

        def p_block(b, carry):
            boff = b * (K * PB)
            iv = idxt_v[pl.ds(pl.multiple_of(boff, PB), LANES)]
            ax = plsc.load_gather(px_v, [iv])
            ay = plsc.load_gather(py_v, [iv])
            az = plsc.load_gather(pz_v, [iv])
            for k in range(1, K):
                iv = idxt_v[pl.ds(pl.multiple_of(boff + k * PB, PB), LANES)]
                ax = jnp.maximum(ax, plsc.load_gather(px_v, [iv]))
                ay = jnp.maximum(ay, plsc.load_gather(py_v, [iv]))
                az = jnp.maximum(az, plsc.load_gather(pz_v, [iv]))
            off = pl.multiple_of(b * PB, PB)
            mpt[0, pl.ds(off, LANES)] = ax
            mpt[1, pl.ds(off, LANES)] = ay
            mpt[2, pl.ds(off, LANES)] = az
            return carry

        lax.fori_loop(0, NB, p_block, 0, unroll=False)
        pltpu.sync_copy(mpt, maxp_hbm.at[wid])

        for i in range(NBUF):
            pltpu.make_async_copy(maxx_hbm.at[pl.ds(0, GPTS)], bufs[i][2],
                                  bufs[i][3]).wait()

    return sc_kernel(idx_flat, idx_t, x, px, py, pz)


def _tc_mlp_bn(maxp_t, p_t, maxx, Wx, Wf, gamma, beta):
    def body(maxpt_ref, pt_ref, maxx_ref, wx_ref, wf_ref, g_ref, b_ref, out_ref):
        h = jnp.dot(maxx_ref[...].astype(jnp.float32), wf_ref[...],
                    preferred_element_type=jnp.float32)
        for cc in range(3):
            d = (maxpt_ref[cc, :] - pt_ref[cc, :]).reshape(N, 1)
            h = h + d * wx_ref[cc, :].reshape(1, OUT)
        mean = jnp.mean(h, axis=0, keepdims=True)
        hc = h - mean
        var = jnp.mean(hc * hc, axis=0, keepdims=True)
        y = hc * lax.rsqrt(var + EPS) * g_ref[...] + b_ref[...]
        out_ref[...] = jnp.maximum(y, 0.0)

    return pl.pallas_call(
        body,
        out_shape=jax.ShapeDtypeStruct((N, OUT), jnp.float32),
    )(maxp_t, p_t, maxx, Wx, Wf, gamma, beta)


def kernel(p, x, o, neighbor_idx, W, gamma, beta):
    del o
    idx = neighbor_idx.astype(jnp.int32)
    idx_pad = jnp.pad(idx, ((0, NPAD - N), (0, 0)))
    idx_flat = idx_pad.reshape(-1)
    idx_t = idx_pad.reshape(NPAD // PB, PB, K).transpose(0, 2, 1).reshape(-1)
    p_t = p.T
    p_pad = jnp.pad(p_t, ((0, 0), (0, NPAD - N)))
    maxx, maxp = _sc_gather_max(idx_flat, idx_t, x.astype(jnp.bfloat16),
                                p_pad[0], p_pad[1], p_pad[2])
    maxp_t = maxp.transpose(1, 0, 2).reshape(3, NPAD)[:, :N]
    return _tc_mlp_bn(maxp_t, p_t, maxx[:N], W[:3], W[3:],
                      gamma[None, :], beta[None, :])


# speedup vs baseline: 6.1554x; 1.0286x over previous
import functools

import jax
import jax.numpy as jnp
from jax import lax
from jax.experimental import pallas as pl
from jax.experimental.pallas import tpu as pltpu
from jax.experimental.pallas import tpu_sc as plsc

N = 10000
K = 32
C = 128
OUT = 128
EPS = 1e-5

NW = 32
PB = 16
PPW = 320
NPAD = NW * PPW
NB = PPW // PB
KC = 8
GPTS = 4
LANES = 16


def _sc_gather_max(idx_flat, x, px, py, pz):
    mesh = plsc.VectorSubcoreMesh(core_axis_name="c", subcore_axis_name="s")

    @functools.partial(
        pl.kernel,
        mesh=mesh,
        compiler_params=pltpu.CompilerParams(
            needs_layout_passes=False, use_tc_tiling_on_sc=False),
        out_type=[
            jax.ShapeDtypeStruct((NPAD, C), jnp.bfloat16),
            jax.ShapeDtypeStruct((NW, 3, PPW), jnp.float32),
        ],
        scratch_types=[
            pltpu.VMEM((PPW * K,), jnp.int32),
            pltpu.VMEM((NPAD,), jnp.float32),
            pltpu.VMEM((NPAD,), jnp.float32),
            pltpu.VMEM((NPAD,), jnp.float32),
            pltpu.VMEM((GPTS * K, C), jnp.bfloat16),
            pltpu.VMEM((GPTS * K, C), jnp.bfloat16),
            pltpu.VMEM((GPTS * K, C), jnp.bfloat16),
            pltpu.VMEM((GPTS * K, C), jnp.bfloat16),
            pltpu.VMEM((GPTS, C), jnp.bfloat16),
            pltpu.VMEM((GPTS, C), jnp.bfloat16),
            pltpu.VMEM((GPTS, C), jnp.bfloat16),
            pltpu.VMEM((GPTS, C), jnp.bfloat16),
            pltpu.VMEM((3, PPW), jnp.float32),
            pltpu.SemaphoreType.DMA,
            pltpu.SemaphoreType.DMA,
            pltpu.SemaphoreType.DMA,
            pltpu.SemaphoreType.DMA,
            pltpu.SemaphoreType.DMA,
            pltpu.SemaphoreType.DMA,
            pltpu.SemaphoreType.DMA,
            pltpu.SemaphoreType.DMA,
            pltpu.SemaphoreType.DMA,
        ],
    )
    def sc_kernel(idx_hbm, x_hbm, px_hbm, py_hbm, pz_hbm,
                  maxx_hbm, maxp_hbm,
                  idx_v, px_v, py_v, pz_v, xg0, xg1, xg2, xg3,
                  mx0, mx1, mx2, mx3, mpt,
                  gsem0, gsem1, gsem2, gsem3, fsem0, fsem1, fsem2, fsem3,
                  psem):
        wid = lax.axis_index("s") * 2 + lax.axis_index("c")
        base_pt = wid * PPW
        pltpu.sync_copy(idx_hbm.at[pl.ds(wid * (PPW * K), PPW * K)], idx_v)
        pltpu.async_copy(px_hbm, px_v, psem)
        pltpu.async_copy(py_hbm, py_v, psem)
        pltpu.async_copy(pz_hbm, pz_v, psem)

        T = PPW // GPTS
        NBUF = 4
        bufs = ((xg0, gsem0, mx0, fsem0), (xg1, gsem1, mx1, fsem1),
                (xg2, gsem2, mx2, fsem2), (xg3, gsem3, mx3, fsem3))

        def issue(t, xg, gsem):
            coff = pl.multiple_of(t * (GPTS * K), GPTS * K)
            pltpu.async_copy(x_hbm.at[idx_v.at[pl.ds(coff, GPTS * K)]],
                             xg, gsem)

        for i in range(NBUF):
            issue(i, bufs[i][0], bufs[i][1])

        def step(t, xg, gsem, mxb, fsem):
            pltpu.make_async_copy(x_hbm.at[pl.ds(0, GPTS * K)], xg, gsem).wait()

            @pl.when(t >= NBUF)
            def _drain_flush():
                pltpu.make_async_copy(maxx_hbm.at[pl.ds(0, GPTS)], mxb,
                                      fsem).wait()

            def col(c, carry):
                off = pl.multiple_of(c * (2 * LANES), 2 * LANES)
                for pt in range(GPTS):
                    vals = [xg[pt * K + k, pl.ds(off, 2 * LANES)]
                            for k in range(K)]
                    while len(vals) > 1:
                        vals = [jnp.maximum(vals[i], vals[i + 1])
                                for i in range(0, len(vals), 2)]
                    mxb[pt, pl.ds(off, 2 * LANES)] = vals[0]
                return carry

            lax.fori_loop(0, C // (2 * LANES), col, 0, unroll=False)
            pltpu.async_copy(mxb, maxx_hbm.at[pl.ds(base_pt + t * GPTS, GPTS)],
                             fsem)

            @pl.when(t + NBUF < T)
            def _next():
                issue(t + NBUF, xg, gsem)

        def rotation(t4, carry):
            for par in range(NBUF):
                step(t4 * NBUF + par, *bufs[par])
            return carry

        lax.fori_loop(0, T // NBUF, rotation, 0, unroll=False)

        pltpu.make_async_copy(px_hbm, px_v, psem).wait()
        pltpu.make_async_copy(py_hbm, py_v, psem).wait()
        pltpu.make_async_copy(pz_hbm, pz_v, psem).wait()
        lane_addr = jax.lax.iota(jnp.int32, LANES) * K

        def p_block(b, carry):
            boff = b * (PB * K)
            iv = plsc.load_gather(idx_v, [lane_addr + boff])
            ax = plsc.load_gather(px_v, [iv])
            ay = plsc.load_gather(py_v, [iv])
            az = plsc.load_gather(pz_v, [iv])
            for k in range(1, K):
                iv = plsc.load_gather(idx_v, [lane_addr + (boff + k)])
                ax = jnp.maximum(ax, plsc.load_gather(px_v, [iv]))
                ay = jnp.maximum(ay, plsc.load_gather(py_v, [iv]))
                az = jnp.maximum(az, plsc.load_gather(pz_v, [iv]))
            off = pl.multiple_of(b * PB, PB)
            mpt[0, pl.ds(off, LANES)] = ax
            mpt[1, pl.ds(off, LANES)] = ay
            mpt[2, pl.ds(off, LANES)] = az
            return carry

        lax.fori_loop(0, NB, p_block, 0, unroll=False)
        pltpu.sync_copy(mpt, maxp_hbm.at[wid])

        for i in range(NBUF):
            pltpu.make_async_copy(maxx_hbm.at[pl.ds(0, GPTS)], bufs[i][2],
                                  bufs[i][3]).wait()

    return sc_kernel(idx_flat, x, px, py, pz)


def _tc_mlp_bn(maxp_t, p_t, maxx, Wx, Wf, gamma, beta):
    def body(maxpt_ref, pt_ref, maxx_ref, wx_ref, wf_ref, g_ref, b_ref, out_ref):
        h = jnp.dot(maxx_ref[...].astype(jnp.float32), wf_ref[...],
                    preferred_element_type=jnp.float32)
        for cc in range(3):
            d = (maxpt_ref[cc, :] - pt_ref[cc, :]).reshape(N, 1)
            h = h + d * wx_ref[cc, :].reshape(1, OUT)
        mean = jnp.mean(h, axis=0, keepdims=True)
        hc = h - mean
        var = jnp.mean(hc * hc, axis=0, keepdims=True)
        y = hc * lax.rsqrt(var + EPS) * g_ref[...] + b_ref[...]
        out_ref[...] = jnp.maximum(y, 0.0)

    return pl.pallas_call(
        body,
        out_shape=jax.ShapeDtypeStruct((N, OUT), jnp.float32),
    )(maxp_t, p_t, maxx, Wx, Wf, gamma, beta)


def kernel(p, x, o, neighbor_idx, W, gamma, beta):
    del o
    idx = neighbor_idx.astype(jnp.int32)
    idx_pad = jnp.pad(idx, ((0, NPAD - N), (0, 0)))
    idx_flat = idx_pad.reshape(-1)
    p_t = p.T
    p_pad = jnp.pad(p_t, ((0, 0), (0, NPAD - N)))
    maxx, maxp = _sc_gather_max(idx_flat, x.astype(jnp.bfloat16),
                                p_pad[0], p_pad[1], p_pad[2])
    maxp_t = maxp.transpose(1, 0, 2).reshape(3, NPAD)[:, :N]
    return _tc_mlp_bn(maxp_t, p_t, maxx[:N], W[:3], W[3:],
                      gamma[None, :], beta[None, :])
